# baseline (speedup 1.0000x reference)
import functools

import jax
import jax.numpy as jnp
from jax import lax
from jax.experimental import pallas as pl
from jax.experimental.pallas import tpu as pltpu
from jax.experimental.pallas import tpu_sc as plsc

N = 1_000_000
D = 16
B = 16384

NC = 2
NS = 16
NW = NC * NS
BW = B // NW
ROWS = 3 * BW
CHUNK = 128
NCHUNK = ROWS // CHUNK

LN2 = 0.6931471805599453
_P = (
    5.06533310e-06,
    1.44239548e+00,
    -7.16986875e-01,
    4.53856241e-01,
    -2.72353158e-01,
    1.17905183e-01,
    -2.48256066e-02,
)

_mesh = plsc.VectorSubcoreMesh(core_axis_name="c", subcore_axis_name="s")


def _softplus16(x):
    y = 1.0 + jnp.exp(x)
    bits = lax.bitcast_convert_type(y, jnp.int32)
    ex = jnp.right_shift(bits, 23) - 127
    m = lax.bitcast_convert_type(
        jnp.bitwise_or(jnp.bitwise_and(bits, 0x007FFFFF), 0x3F800000),
        jnp.float32,
    )
    t = m - 1.0
    p = jnp.float32(_P[6])
    for k in (5, 4, 3, 2, 1, 0):
        p = p * t + jnp.float32(_P[k])
    log2y = ex.astype(jnp.float32) + p
    return log2y * jnp.float32(LN2)


@functools.partial(
    pl.kernel,
    mesh=_mesh,
    out_type=jax.ShapeDtypeStruct((B,), jnp.float32),
    compiler_params=pltpu.CompilerParams(
        needs_layout_passes=False, use_tc_tiling_on_sc=False
    ),
    scratch_types=[
        pltpu.VMEM((NCHUNK, CHUNK), jnp.int32),
        pltpu.VMEM((ROWS, D), jnp.float32),
        pltpu.VMEM((BW,), jnp.float32),
        pltpu.SemaphoreType.DMA,
    ],
)
def _ste_sc(idx_hbm, table_hbm, out_hbm, idx_v, rows_v, out_v, sem):
    wid = lax.axis_index("s") * NC + lax.axis_index("c")
    base = wid * BW

    pltpu.sync_copy(idx_hbm.at[wid], idx_v)

    copies = []
    for k in range(NCHUNK):
        copies.append(
            pltpu.async_copy(
                table_hbm.at[idx_v.at[k]],
                rows_v.at[pl.ds(k * CHUNK, CHUNK)],
                sem,
            )
        )
    for cp in copies:
        cp.wait()

    lane = lax.iota(jnp.int32, 16)

    def group(g, carry):
        r0 = g * 48
        acc = jnp.zeros((16,), jnp.float32)
        for j in range(16):
            h = rows_v[r0 + 3 * j, :]
            w = rows_v[r0 + 3 * j + 1, :]
            l = rows_v[r0 + 3 * j + 2, :]
            dw = h - w
            dl = h - l
            v = dw * dw - dl * dl
            s = jnp.sum(v)
            acc = jnp.where(lane == j, s, acc)
        out_v[pl.ds(g * 16, 16)] = _softplus16(acc)
        return carry

    lax.fori_loop(0, BW // 16, group, 0)
    pltpu.sync_copy(out_v, out_hbm.at[pl.ds(base, BW)])


def kernel(h_w_l, embedding):
    idx = h_w_l.reshape(NW, NCHUNK, CHUNK)
    return _ste_sc(idx, embedding)
